# baseline (speedup 1.0000x reference)
import functools

import jax
import jax.numpy as jnp
from jax import lax
from jax.experimental import pallas as pl
from jax.experimental.pallas import tpu as pltpu
from jax.experimental.pallas import tpu_sc as plsc

_BLOCK_S = 512

_CHUNK_ROWS = 32


def _add_pe_kernel(x_ref, pe_ref, o_ref):
    o_ref[...] = x_ref[...] + pe_ref[...][None, :, :]


def _kernel_tc(x, pos_table):
    batch, seq_len, d_model = x.shape
    block_s = _BLOCK_S if seq_len % _BLOCK_S == 0 else seq_len
    grid = (seq_len // block_s,)
    return pl.pallas_call(
        _add_pe_kernel,
        grid=grid,
        in_specs=[
            pl.BlockSpec((batch, block_s, d_model), lambda s: (0, s, 0)),
            pl.BlockSpec((block_s, d_model), lambda s: (s, 0)),
        ],
        out_specs=pl.BlockSpec((batch, block_s, d_model), lambda s: (0, s, 0)),
        out_shape=jax.ShapeDtypeStruct(x.shape, x.dtype),
    )(x, pos_table[:seq_len])


def _make_sc_kernel(batch, seq_len, d_model, dtype):
    info = plsc.get_sparse_core_info()
    n_workers = info.num_cores * info.num_subcores
    rows_per_worker = seq_len // n_workers
    chunk = 8
    n_gens = rows_per_worker // chunk
    nbuf = 3

    mesh = plsc.VectorSubcoreMesh(core_axis_name="c", subcore_axis_name="s")

    @functools.partial(
        pl.kernel,
        mesh=mesh,
        out_type=jax.ShapeDtypeStruct((batch, seq_len, d_model), dtype),
        compiler_params=pltpu.CompilerParams(use_tc_tiling_on_sc=True),
        scratch_types=[
            pltpu.VMEM((nbuf, batch, chunk, d_model), dtype),
            pltpu.VMEM((nbuf, chunk, d_model), dtype),
        ] + [pltpu.SemaphoreType.DMA] * (2 * nbuf),
    )
    def sc_kernel(x_hbm, pe_hbm, out_hbm, xv, pev, *sems):
        wid = lax.axis_index("s") * info.num_cores + lax.axis_index("c")
        base = wid * rows_per_worker
        in_sems = sems[:nbuf]
        out_sems = sems[nbuf:]

        def start_gen(g):
            p = g % nbuf
            r0 = base + g * chunk
            h = [pltpu.async_copy(
                pe_hbm.at[pl.ds(r0, chunk)], pev.at[p], in_sems[p])]
            for b in range(batch):
                h.append(pltpu.async_copy(
                    x_hbm.at[b, pl.ds(r0, chunk)], xv.at[p, b], in_sems[p]))
            return h

        pending_in = {0: start_gen(0), 1: start_gen(1)}
        pending_out = {}
        for g in range(n_gens):
            p = g % nbuf
            hs = pending_in.pop(g)
            hs[0].wait()
            for b in range(batch):
                hs[1 + b].wait()

                def add_body(j, _, b=b):
                    sl = pl.ds(j * 16, 16)
                    for r in range(chunk):
                        xv[p, b, r, sl] = xv[p, b, r, sl] + pev[p, r, sl]
                    return 0

                lax.fori_loop(0, d_model // 16, add_body, 0)

            r0 = base + g * chunk
            pending_out[g] = [
                pltpu.async_copy(
                    xv.at[p], out_hbm.at[:, pl.ds(r0, chunk)], out_sems[p])]
            nxt = g + 2
            if nxt < n_gens:
                if g >= 1:
                    for h in pending_out.pop(g - 1):
                        h.wait()
                pending_in[nxt] = start_gen(nxt)
        for g in sorted(pending_out):
            for h in pending_out[g]:
                h.wait()

    return sc_kernel


def kernel(x, pos_table):
    batch, seq_len, d_model = x.shape
    sc = _make_sc_kernel(batch, seq_len, d_model, x.dtype)
    return sc(x, pos_table[:seq_len])


# speedup vs baseline: 1.0047x; 1.0047x over previous
import functools

import jax
import jax.numpy as jnp
from jax import lax
from jax.experimental import pallas as pl
from jax.experimental.pallas import tpu as pltpu
from jax.experimental.pallas import tpu_sc as plsc


def _make_sc_kernel(batch, seq_len, d_model, dtype):
    info = plsc.get_sparse_core_info()
    n_workers = info.num_cores * info.num_subcores
    rows_per_worker = seq_len // n_workers
    chunk = 8
    n_gens = rows_per_worker // chunk
    nbuf = 3

    mesh = plsc.VectorSubcoreMesh(core_axis_name="c", subcore_axis_name="s")

    @functools.partial(
        pl.kernel,
        mesh=mesh,
        out_type=jax.ShapeDtypeStruct((batch, seq_len, d_model), dtype),
        compiler_params=pltpu.CompilerParams(use_tc_tiling_on_sc=True),
        scratch_types=[
            pltpu.VMEM((nbuf, batch, chunk, d_model), dtype),
            pltpu.VMEM((nbuf, chunk, d_model), dtype),
        ] + [pltpu.SemaphoreType.DMA] * (2 * nbuf),
    )
    def sc_kernel(x_hbm, pe_hbm, out_hbm, xv, pev, *sems):
        wid = lax.axis_index("s") * info.num_cores + lax.axis_index("c")
        base = wid * rows_per_worker
        in_sems = sems[:nbuf]
        out_sems = sems[nbuf:]

        def start_gen(g):
            p = g % nbuf
            r0 = base + g * chunk
            return [
                pltpu.async_copy(
                    pe_hbm.at[pl.ds(r0, chunk)], pev.at[p], in_sems[p]),
                pltpu.async_copy(
                    x_hbm.at[:, pl.ds(r0, chunk)], xv.at[p], in_sems[p]),
            ]

        pending_in = {g: start_gen(g) for g in range(min(2, n_gens))}
        pending_out = {}
        for g in range(n_gens):
            p = g % nbuf
            for h in pending_in.pop(g):
                h.wait()

            def add_body(j, _):
                sl = pl.ds(j * 16, 16)
                for r in range(chunk):
                    pe_vec = pev[p, r, sl]
                    for b in range(batch):
                        xv[p, b, r, sl] = xv[p, b, r, sl] + pe_vec
                return 0

            lax.fori_loop(0, d_model // 16, add_body, 0)

            r0 = base + g * chunk
            pending_out[g] = [
                pltpu.async_copy(
                    xv.at[p], out_hbm.at[:, pl.ds(r0, chunk)], out_sems[p])]
            nxt = g + 2
            if nxt < n_gens:
                if g >= 1:
                    for h in pending_out.pop(g - 1):
                        h.wait()
                pending_in[nxt] = start_gen(nxt)
        for g in sorted(pending_out):
            for h in pending_out[g]:
                h.wait()

    return sc_kernel


def kernel(x, pos_table):
    batch, seq_len, d_model = x.shape
    sc = _make_sc_kernel(batch, seq_len, d_model, x.dtype)
    return sc(x, pos_table[:seq_len])


# speedup vs baseline: 1.0255x; 1.0207x over previous
import functools

import jax
import jax.numpy as jnp
from jax import lax
from jax.experimental import pallas as pl
from jax.experimental.pallas import tpu as pltpu
from jax.experimental.pallas import tpu_sc as plsc


def _make_sc_kernel(batch, seq_len, d_model, dtype):
    info = plsc.get_sparse_core_info()
    n_workers = info.num_cores * info.num_subcores
    rows_per_worker = seq_len // n_workers
    chunk = 8
    n_gens = rows_per_worker // chunk
    nbuf = 3

    mesh = plsc.VectorSubcoreMesh(core_axis_name="c", subcore_axis_name="s")

    @functools.partial(
        pl.kernel,
        mesh=mesh,
        out_type=jax.ShapeDtypeStruct((batch, seq_len, d_model), dtype),
        compiler_params=pltpu.CompilerParams(use_tc_tiling_on_sc=True),
        scratch_types=[
            pltpu.VMEM((nbuf, batch, chunk, d_model), dtype),
            pltpu.VMEM((nbuf, chunk, d_model), dtype),
        ] + [pltpu.SemaphoreType.DMA] * (2 * nbuf),
    )
    def sc_kernel(x_hbm, pe_hbm, out_hbm, xv, pev, *sems):
        wid = lax.axis_index("s") * info.num_cores + lax.axis_index("c")
        in_sems = sems[:nbuf]
        out_sems = sems[nbuf:]

        def start_gen(g):
            p = g % nbuf
            r0 = (g * n_workers + wid) * chunk
            return [
                pltpu.async_copy(
                    pe_hbm.at[pl.ds(r0, chunk)], pev.at[p], in_sems[p]),
                pltpu.async_copy(
                    x_hbm.at[:, pl.ds(r0, chunk)], xv.at[p], in_sems[p]),
            ]

        pending_in = {g: start_gen(g) for g in range(min(2, n_gens))}
        pending_out = {}
        for g in range(n_gens):
            p = g % nbuf
            for h in pending_in.pop(g):
                h.wait()

            def add_body(j, _):
                sl = pl.ds(j * 16, 16)
                for r in range(chunk):
                    pe_vec = pev[p, r, sl]
                    for b in range(batch):
                        xv[p, b, r, sl] = xv[p, b, r, sl] + pe_vec
                return 0

            lax.fori_loop(0, d_model // 16, add_body, 0)

            r0 = (g * n_workers + wid) * chunk
            pending_out[g] = [
                pltpu.async_copy(
                    xv.at[p], out_hbm.at[:, pl.ds(r0, chunk)], out_sems[p])]
            nxt = g + 2
            if nxt < n_gens:
                if g >= 1:
                    for h in pending_out.pop(g - 1):
                        h.wait()
                pending_in[nxt] = start_gen(nxt)
        for g in sorted(pending_out):
            for h in pending_out[g]:
                h.wait()

    return sc_kernel


def kernel(x, pos_table):
    batch, seq_len, d_model = x.shape
    sc = _make_sc_kernel(batch, seq_len, d_model, x.dtype)
    return sc(x, pos_table[:seq_len])


# speedup vs baseline: 1.0264x; 1.0009x over previous
import functools

import jax
import jax.numpy as jnp
from jax import lax
from jax.experimental import pallas as pl
from jax.experimental.pallas import tpu as pltpu
from jax.experimental.pallas import tpu_sc as plsc


def _make_sc_kernel(batch, seq_len, d_model, dtype):
    info = plsc.get_sparse_core_info()
    n_workers = info.num_cores * info.num_subcores
    rows_per_worker = seq_len // n_workers
    chunk = 8
    n_gens = rows_per_worker // chunk
    nbuf = 3

    mesh = plsc.VectorSubcoreMesh(core_axis_name="c", subcore_axis_name="s")

    @functools.partial(
        pl.kernel,
        mesh=mesh,
        out_type=jax.ShapeDtypeStruct((batch, seq_len, d_model), dtype),
        compiler_params=pltpu.CompilerParams(use_tc_tiling_on_sc=True),
        scratch_types=[
            pltpu.VMEM((nbuf, batch, chunk, d_model), dtype),
            pltpu.VMEM((nbuf, chunk, d_model), dtype),
        ] + [pltpu.SemaphoreType.DMA] * (2 * nbuf),
    )
    def sc_kernel(x_hbm, pe_hbm, out_hbm, xv, pev, *sems):
        wid = lax.axis_index("c") * info.num_subcores + lax.axis_index("s")
        in_sems = sems[:nbuf]
        out_sems = sems[nbuf:]

        def start_gen(g):
            p = g % nbuf
            r0 = (g * n_workers + wid) * chunk
            return [
                pltpu.async_copy(
                    pe_hbm.at[pl.ds(r0, chunk)], pev.at[p], in_sems[p]),
                pltpu.async_copy(
                    x_hbm.at[:, pl.ds(r0, chunk)], xv.at[p], in_sems[p]),
            ]

        pending_in = {g: start_gen(g) for g in range(min(2, n_gens))}
        pending_out = {}
        for g in range(n_gens):
            p = g % nbuf
            for h in pending_in.pop(g):
                h.wait()

            def add_body(j, _):
                sl = pl.ds(j * 16, 16)
                for r in range(chunk):
                    pe_vec = pev[p, r, sl]
                    for b in range(batch):
                        xv[p, b, r, sl] = xv[p, b, r, sl] + pe_vec
                return 0

            lax.fori_loop(0, d_model // 16, add_body, 0)

            r0 = (g * n_workers + wid) * chunk
            pending_out[g] = [
                pltpu.async_copy(
                    xv.at[p], out_hbm.at[:, pl.ds(r0, chunk)], out_sems[p])]
            nxt = g + 2
            if nxt < n_gens:
                if g >= 1:
                    for h in pending_out.pop(g - 1):
                        h.wait()
                pending_in[nxt] = start_gen(nxt)
        for g in sorted(pending_out):
            for h in pending_out[g]:
                h.wait()

    return sc_kernel


def kernel(x, pos_table):
    batch, seq_len, d_model = x.shape
    sc = _make_sc_kernel(batch, seq_len, d_model, x.dtype)
    return sc(x, pos_table[:seq_len])


# speedup vs baseline: 1.0270x; 1.0006x over previous
import functools

import jax
from jax import lax
from jax.experimental import pallas as pl
from jax.experimental.pallas import tpu as pltpu
from jax.experimental.pallas import tpu_sc as plsc


def _make_sc_kernel(batch, seq_len, d_model, dtype):
    info = plsc.get_sparse_core_info()
    n_workers = info.num_cores * info.num_subcores
    rows_per_worker = seq_len // n_workers
    chunk = 8
    n_gens = rows_per_worker // chunk
    nbuf = 3

    mesh = plsc.VectorSubcoreMesh(core_axis_name="c", subcore_axis_name="s")

    @functools.partial(
        pl.kernel,
        mesh=mesh,
        out_type=jax.ShapeDtypeStruct((batch, seq_len, d_model), dtype),
        compiler_params=pltpu.CompilerParams(use_tc_tiling_on_sc=True),
        scratch_types=[
            pltpu.VMEM((nbuf, batch, chunk, d_model), dtype),
            pltpu.VMEM((nbuf, chunk, d_model), dtype),
        ] + [pltpu.SemaphoreType.DMA] * (2 * nbuf),
    )
    def sc_kernel(x_hbm, pe_hbm, out_hbm, xv, pev, *sems):
        wid = lax.axis_index("c") * info.num_subcores + lax.axis_index("s")
        in_sems = sems[:nbuf]
        out_sems = sems[nbuf:]

        def start_gen(g):
            p = g % nbuf
            r0 = (g * n_workers + wid) * chunk
            return [
                pltpu.async_copy(
                    pe_hbm.at[pl.ds(r0, chunk)], pev.at[p], in_sems[p]),
                pltpu.async_copy(
                    x_hbm.at[:, pl.ds(r0, chunk)], xv.at[p], in_sems[p]),
            ]

        pending_in = {g: start_gen(g) for g in range(min(2, n_gens))}
        pending_out = {}
        for g in range(n_gens):
            p = g % nbuf
            for h in pending_in.pop(g):
                h.wait()

            def add_body(j, _):
                sl = pl.ds(j * 16, 16)
                for r in range(chunk):
                    pe_vec = pev[p, r, sl]
                    for b in range(batch):
                        xv[p, b, r, sl] = xv[p, b, r, sl] + pe_vec
                return 0

            lax.fori_loop(0, d_model // 16, add_body, 0)

            r0 = (g * n_workers + wid) * chunk
            pending_out[g] = [
                pltpu.async_copy(
                    xv.at[p], out_hbm.at[:, pl.ds(r0, chunk)], out_sems[p])]
            nxt = g + 2
            if nxt < n_gens:
                if g >= 1:
                    for h in pending_out.pop(g - 1):
                        h.wait()
                pending_in[nxt] = start_gen(nxt)
        for g in sorted(pending_out):
            for h in pending_out[g]:
                h.wait()

    return sc_kernel


def kernel(x, pos_table):
    batch, seq_len, d_model = x.shape
    sc = _make_sc_kernel(batch, seq_len, d_model, x.dtype)
    return sc(x, pos_table[:seq_len])
